# Initial kernel scaffold; baseline (speedup 1.0000x reference)
import jax
import jax.numpy as jnp
from jax.experimental import pallas as pl


def kernel(x, WA, bA, WB, bB, WV, bV):
    raise NotImplementedError("write your pallas kernel here")


# speedup vs baseline: 2.5030x; 2.5030x over previous
import jax
import jax.numpy as jnp
from jax.experimental import pallas as pl
from jax.experimental.pallas import tpu as pltpu

_CM = 96
_CN = 96


def _body(x_ref, w_ref, b_ref, o_ref):
    X = x_ref[0]
    P = jnp.dot(w_ref[...], X, preferred_element_type=jnp.float32) + b_ref[...]
    A = P[0:_CM]
    B = P[_CM:_CM + _CN]
    V = P[_CM + _CN:_CM + 2 * _CN]

    Be = jnp.exp(B - jnp.max(B, axis=1, keepdims=True))
    sB = Be / jnp.sum(Be, axis=1, keepdims=True)
    Ve = jnp.exp(V - jnp.max(V, axis=0, keepdims=True))
    sV = Ve / jnp.sum(Ve, axis=0, keepdims=True)

    G = jax.lax.dot_general(A, sB, (((1,), (1,)), ((), ())),
                            preferred_element_type=jnp.float32)
    o_ref[0] = jnp.dot(G, sV, preferred_element_type=jnp.float32)


def kernel(x, WA, bA, WB, bB, WV, bV):
    b, c, d, h, w = x.shape
    dhw = d * h * w
    x2 = x.reshape(b, c, dhw)
    W = jnp.concatenate([WA, WB, WV], axis=0)
    bias = jnp.concatenate([bA, bB, bV], axis=0)[:, None]

    out = pl.pallas_call(
        _body,
        grid=(b,),
        in_specs=[
            pl.BlockSpec((1, c, dhw), lambda i: (i, 0, 0)),
            pl.BlockSpec((3 * _CN, c), lambda i: (0, 0)),
            pl.BlockSpec((3 * _CN, 1), lambda i: (0, 0)),
        ],
        out_specs=pl.BlockSpec((1, _CM, dhw), lambda i: (i, 0, 0)),
        out_shape=jax.ShapeDtypeStruct((b, _CM, dhw), jnp.float32),
        compiler_params=pltpu.CompilerParams(
            dimension_semantics=("parallel",),
        ),
    )(x2, W, bias)
    return out.reshape(b, _CM, d, h, w)
